# baseline (speedup 1.0000x reference)
import jax
import jax.numpy as jnp
from jax import lax
from jax.experimental import pallas as pl
from jax.experimental.pallas import tpu as pltpu
from jax.experimental.pallas import tpu_sc as plsc

_NUM_TASKS = 8
_NUM_MODALITIES = 4
_PRIOR_LEN = 16
_EMBED_DIM = 768
_BATCH = 4096

_ROW = _PRIOR_LEN * _EMBED_DIM
_NROWS = 2 * _BATCH
_NC, _NS = 2, 16
_NW = _NC * _NS
_NG = _NW // 2
_B_PER_G = _BATCH // _NG
_K = 16


def _body(table_hbm, idx_hbm, out_hbm, tbl_v, idx_v, sem):
    wid = lax.axis_index("s") * _NC + lax.axis_index("c")
    half = wid % 2
    base = (wid // 2) * _B_PER_G
    pltpu.sync_copy(idx_hbm.at[wid], idx_v)

    off = half * (_NUM_TASKS * _ROW)

    def _row_copy(i, r):
        p = (base + i) * 2 + half
        pltpu.async_copy(table_hbm.at[pl.ds(off + r * _ROW, _ROW)],
                         out_hbm.at[pl.ds(p * _ROW, _ROW)], sem)

    def _wait_row():
        pltpu.make_async_copy(table_hbm.at[pl.ds(0, _ROW)],
                              out_hbm.at[pl.ds(0, _ROW)], sem).wait()

    rows0 = idx_v[pl.ds(0, _K)]
    for k in range(_K):
        _row_copy(k, rows0[k])

    @pl.loop(_K, _B_PER_G, step=_K)
    def _block(i0):
        rows = idx_v[pl.ds(i0, _K)]
        for k in range(_K):
            _wait_row()
            _row_copy(i0 + k, rows[k])

    for _ in range(_K):
        _wait_row()


_sc_gather = pl.kernel(
    _body,
    out_type=jax.ShapeDtypeStruct((_NROWS * _ROW,), jnp.float32),
    mesh=plsc.VectorSubcoreMesh(
        core_axis_name="c", subcore_axis_name="s",
        num_cores=_NC, num_subcores=_NS,
    ),
    scratch_types=[
        pltpu.VMEM((_NUM_TASKS * _ROW,), jnp.float32),
        pltpu.VMEM((_B_PER_G,), jnp.int32),
        pltpu.SemaphoreType.DMA,
    ],
)


def kernel(task_table, modality_table, task_idx, modality_idx):
    table = jnp.concatenate(
        [task_table.reshape(_NUM_TASKS * _ROW),
         modality_table.reshape(_NUM_MODALITIES * _ROW)])
    idx = jnp.stack(
        [task_idx.astype(jnp.int32).reshape(_NG, _B_PER_G),
         modality_idx.astype(jnp.int32).reshape(_NG, _B_PER_G)], axis=1)
    idx = idx.reshape(_NW, _B_PER_G)
    out = _sc_gather(table, idx)
    return out.reshape(_BATCH, 2 * _PRIOR_LEN, _EMBED_DIM)


# speedup vs baseline: 22.5407x; 22.5407x over previous
import jax
import jax.numpy as jnp
from jax import lax
from jax.experimental import pallas as pl
from jax.experimental.pallas import tpu as pltpu
from jax.experimental.pallas import tpu_sc as plsc

_NUM_TASKS = 8
_NUM_MODALITIES = 4
_PRIOR_LEN = 16
_EMBED_DIM = 768
_BATCH = 4096

_ROW = _PRIOR_LEN * _EMBED_DIM
_NROWS = 2 * _BATCH
_NC, _NS = 2, 16
_NW = _NC * _NS
_NG = _NW // 2

_B_SC = 1536
_B_PER_G = _B_SC // _NG
_K = 16

_BB = 32
_N_TC_BLOCKS = (_BATCH - _B_SC) // _BB


def _sc_body(table_hbm, idx_hbm, out_hbm, tbl_v, idx_v, sem):
    wid = lax.axis_index("s") * _NC + lax.axis_index("c")
    half = wid % 2
    base = (wid // 2) * _B_PER_G
    pltpu.sync_copy(idx_hbm.at[wid], idx_v)

    @pl.when(half == 0)
    def _():
        pltpu.sync_copy(table_hbm.at[pl.ds(0, _NUM_TASKS * _ROW)], tbl_v)

    @pl.when(half == 1)
    def _():
        pltpu.sync_copy(
            table_hbm.at[pl.ds(_NUM_TASKS * _ROW, _NUM_MODALITIES * _ROW)],
            tbl_v.at[pl.ds(0, _NUM_MODALITIES * _ROW)])

    def _row_copy(i, r):
        p = (base + i) * 2 + half
        pltpu.async_copy(tbl_v.at[pl.ds(r * _ROW, _ROW)],
                         out_hbm.at[pl.ds(p * _ROW, _ROW)], sem)

    def _wait_row():
        pltpu.make_async_copy(tbl_v.at[pl.ds(0, _ROW)],
                              out_hbm.at[pl.ds(0, _ROW)], sem).wait()

    rows0 = idx_v[pl.ds(0, _K)]
    for k in range(_K):
        _row_copy(k, rows0[k])

    @pl.loop(_K, _B_PER_G, step=_K)
    def _block(i0):
        rows = idx_v[pl.ds(i0, _K)]
        for k in range(_K):
            _wait_row()
            _row_copy(i0 + k, rows[k])

    for _ in range(_K):
        _wait_row()


_sc_gather = pl.kernel(
    _sc_body,
    out_type=jax.ShapeDtypeStruct((_NROWS * _ROW,), jnp.float32),
    mesh=plsc.VectorSubcoreMesh(
        core_axis_name="c", subcore_axis_name="s",
        num_cores=_NC, num_subcores=_NS,
    ),
    scratch_types=[
        pltpu.VMEM((_NUM_TASKS * _ROW,), jnp.float32),
        pltpu.VMEM((_B_PER_G,), jnp.int32),
        pltpu.SemaphoreType.DMA,
    ],
)


def _tc_body(sc_ref, ttab_ref, mtab_ref, ti_ref, mi_ref, out_ref):
    del sc_ref
    i = pl.program_id(0)
    b0 = _B_SC + i * _BB
    for b in range(_BB):
        out_ref[b, 0:_PRIOR_LEN, :] = ttab_ref[ti_ref[b0 + b]]
        out_ref[b, _PRIOR_LEN:, :] = mtab_ref[mi_ref[b0 + b]]


_tc_fill = pl.pallas_call(
    _tc_body,
    grid=(_N_TC_BLOCKS,),
    in_specs=[
        pl.BlockSpec(memory_space=pl.ANY),
        pl.BlockSpec((_NUM_TASKS, _PRIOR_LEN, _EMBED_DIM),
                     lambda i: (0, 0, 0)),
        pl.BlockSpec((_NUM_MODALITIES, _PRIOR_LEN, _EMBED_DIM),
                     lambda i: (0, 0, 0)),
        pl.BlockSpec(memory_space=pltpu.SMEM),
        pl.BlockSpec(memory_space=pltpu.SMEM),
    ],
    out_specs=pl.BlockSpec((_BB, 2 * _PRIOR_LEN, _EMBED_DIM),
                           lambda i: (_B_SC // _BB + i, 0, 0)),
    out_shape=jax.ShapeDtypeStruct((_BATCH, 2 * _PRIOR_LEN, _EMBED_DIM),
                                   jnp.float32),
    input_output_aliases={0: 0},
)


def kernel(task_table, modality_table, task_idx, modality_idx):
    table = jnp.concatenate(
        [task_table.reshape(_NUM_TASKS * _ROW),
         modality_table.reshape(_NUM_MODALITIES * _ROW)])
    ti = task_idx.astype(jnp.int32)
    mi = modality_idx.astype(jnp.int32)
    sc_idx = jnp.stack(
        [ti[:_B_SC].reshape(_NG, _B_PER_G),
         mi[:_B_SC].reshape(_NG, _B_PER_G)], axis=1).reshape(_NW, _B_PER_G)
    sc_out = _sc_gather(table, sc_idx)
    sc_out = sc_out.reshape(_BATCH, 2 * _PRIOR_LEN, _EMBED_DIM)
    return _tc_fill(sc_out, task_table, modality_table, ti, mi)


# speedup vs baseline: 79.5444x; 3.5289x over previous
import jax
import jax.numpy as jnp
from jax import lax
from jax.experimental import pallas as pl
from jax.experimental.pallas import tpu as pltpu
from jax.experimental.pallas import tpu_sc as plsc

_NUM_TASKS = 8
_NUM_MODALITIES = 4
_PRIOR_LEN = 16
_EMBED_DIM = 768
_BATCH = 4096

_NC, _NS = 2, 16
_NW = _NC * _NS
_NG = _NW // 2

_B_SC = 1536
_B_PER_G = _B_SC // _NG
_K = 16

_BB = 32
_N_TC_BLOCKS = (_BATCH - _B_SC) // _BB


def _sc_body(table_hbm, idx_hbm, out_hbm, tbl_v, idx_v, sem):
    wid = lax.axis_index("s") * _NC + lax.axis_index("c")
    half = wid % 2
    base = (wid // 2) * _B_PER_G
    loff = half * _PRIOR_LEN
    pltpu.sync_copy(idx_hbm.at[wid], idx_v)

    @pl.when(half == 0)
    def _():
        pltpu.sync_copy(table_hbm.at[pl.ds(0, _NUM_TASKS)], tbl_v)

    @pl.when(half == 1)
    def _():
        pltpu.sync_copy(table_hbm.at[pl.ds(_NUM_TASKS, _NUM_MODALITIES)],
                        tbl_v.at[pl.ds(0, _NUM_MODALITIES)])

    def _row_copy(i, r):
        pltpu.async_copy(
            tbl_v.at[r],
            out_hbm.at[base + i, pl.ds(loff, _PRIOR_LEN)], sem)

    def _wait_row():
        pltpu.make_async_copy(
            tbl_v.at[0],
            out_hbm.at[0, pl.ds(0, _PRIOR_LEN)], sem).wait()

    rows0 = idx_v[pl.ds(0, _K)]
    for k in range(_K):
        _row_copy(k, rows0[k])

    @pl.loop(_K, _B_PER_G, step=_K)
    def _block(i0):
        rows = idx_v[pl.ds(i0, _K)]
        for k in range(_K):
            _wait_row()
            _row_copy(i0 + k, rows[k])

    for _ in range(_K):
        _wait_row()


_sc_gather = pl.kernel(
    _sc_body,
    out_type=jax.ShapeDtypeStruct((_BATCH, 2 * _PRIOR_LEN, _EMBED_DIM),
                                  jnp.float32),
    mesh=plsc.VectorSubcoreMesh(
        core_axis_name="c", subcore_axis_name="s",
        num_cores=_NC, num_subcores=_NS,
    ),
    scratch_types=[
        pltpu.VMEM((_NUM_TASKS, _PRIOR_LEN, _EMBED_DIM), jnp.float32),
        pltpu.VMEM((_B_PER_G,), jnp.int32),
        pltpu.SemaphoreType.DMA,
    ],
)


def _tc_body(sc_ref, ttab_ref, mtab_ref, ti_ref, mi_ref, out_ref):
    del sc_ref
    i = pl.program_id(0)
    b0 = _B_SC + i * _BB
    for b in range(_BB):
        out_ref[b, 0:_PRIOR_LEN, :] = ttab_ref[ti_ref[b0 + b]]
        out_ref[b, _PRIOR_LEN:, :] = mtab_ref[mi_ref[b0 + b]]


_tc_fill = pl.pallas_call(
    _tc_body,
    grid=(_N_TC_BLOCKS,),
    in_specs=[
        pl.BlockSpec(memory_space=pl.ANY),
        pl.BlockSpec((_NUM_TASKS, _PRIOR_LEN, _EMBED_DIM),
                     lambda i: (0, 0, 0)),
        pl.BlockSpec((_NUM_MODALITIES, _PRIOR_LEN, _EMBED_DIM),
                     lambda i: (0, 0, 0)),
        pl.BlockSpec(memory_space=pltpu.SMEM),
        pl.BlockSpec(memory_space=pltpu.SMEM),
    ],
    out_specs=pl.BlockSpec((_BB, 2 * _PRIOR_LEN, _EMBED_DIM),
                           lambda i: (_B_SC // _BB + i, 0, 0)),
    out_shape=jax.ShapeDtypeStruct((_BATCH, 2 * _PRIOR_LEN, _EMBED_DIM),
                                   jnp.float32),
    input_output_aliases={0: 0},
)


def kernel(task_table, modality_table, task_idx, modality_idx):
    table = jnp.concatenate([task_table, modality_table], axis=0)
    ti = task_idx.astype(jnp.int32)
    mi = modality_idx.astype(jnp.int32)
    sc_idx = jnp.stack(
        [ti[:_B_SC].reshape(_NG, _B_PER_G),
         mi[:_B_SC].reshape(_NG, _B_PER_G)], axis=1).reshape(_NW, _B_PER_G)
    sc_out = _sc_gather(table, sc_idx)
    return _tc_fill(sc_out, task_table, modality_table, ti, mi)


# speedup vs baseline: 82.8889x; 1.0420x over previous
import jax
import jax.numpy as jnp
from jax import lax
from jax.experimental import pallas as pl
from jax.experimental.pallas import tpu as pltpu
from jax.experimental.pallas import tpu_sc as plsc

_NUM_TASKS = 8
_NUM_MODALITIES = 4
_PRIOR_LEN = 16
_EMBED_DIM = 768
_BATCH = 4096

_NC, _NS = 2, 16
_NW = _NC * _NS
_NG = _NW // 2
_B_PER_G = _BATCH // _NG
_K = 16


def _sc_body(table_hbm, idx_hbm, out_hbm, tbl_v, idx_v, sem):
    wid = lax.axis_index("s") * _NC + lax.axis_index("c")
    half = wid % 2
    base = (wid // 2) * _B_PER_G
    loff = half * _PRIOR_LEN
    pltpu.sync_copy(idx_hbm.at[wid], idx_v)

    @pl.when(half == 0)
    def _():
        pltpu.sync_copy(table_hbm.at[pl.ds(0, _NUM_TASKS)], tbl_v)

    @pl.when(half == 1)
    def _():
        pltpu.sync_copy(table_hbm.at[pl.ds(_NUM_TASKS, _NUM_MODALITIES)],
                        tbl_v.at[pl.ds(0, _NUM_MODALITIES)])

    def _row_copy(i, r):
        pltpu.async_copy(
            tbl_v.at[r],
            out_hbm.at[base + i, pl.ds(loff, _PRIOR_LEN)], sem)

    def _wait_row():
        pltpu.make_async_copy(
            tbl_v.at[0],
            out_hbm.at[0, pl.ds(0, _PRIOR_LEN)], sem).wait()

    rows0 = idx_v[pl.ds(0, _K)]
    for k in range(_K):
        _row_copy(k, rows0[k])

    @pl.loop(_K, _B_PER_G, step=_K)
    def _block(i0):
        rows = idx_v[pl.ds(i0, _K)]
        for k in range(_K):
            _wait_row()
            _row_copy(i0 + k, rows[k])

    for _ in range(_K):
        _wait_row()


_sc_gather = pl.kernel(
    _sc_body,
    out_type=jax.ShapeDtypeStruct((_BATCH, 2 * _PRIOR_LEN, _EMBED_DIM),
                                  jnp.float32),
    mesh=plsc.VectorSubcoreMesh(
        core_axis_name="c", subcore_axis_name="s",
        num_cores=_NC, num_subcores=_NS,
    ),
    scratch_types=[
        pltpu.VMEM((_NUM_TASKS, _PRIOR_LEN, _EMBED_DIM), jnp.float32),
        pltpu.VMEM((_B_PER_G,), jnp.int32),
        pltpu.SemaphoreType.DMA,
    ],
)


def kernel(task_table, modality_table, task_idx, modality_idx):
    table = jnp.concatenate([task_table, modality_table], axis=0)
    sc_idx = jnp.stack(
        [task_idx.astype(jnp.int32).reshape(_NG, _B_PER_G),
         modality_idx.astype(jnp.int32).reshape(_NG, _B_PER_G)],
        axis=1).reshape(_NW, _B_PER_G)
    return _sc_gather(table, sc_idx)
